# Initial kernel scaffold; baseline (speedup 1.0000x reference)
import jax
import jax.numpy as jnp
from jax.experimental import pallas as pl


def kernel(tl_croped, tl, H):
    raise NotImplementedError("write your pallas kernel here")


# speedup vs baseline: 4.8628x; 4.8628x over previous
import functools

import jax
import jax.numpy as jnp
from jax import lax
from jax.experimental import pallas as pl
from jax.experimental.pallas import tpu as pltpu
from jax.experimental.pallas import tpu_sc as plsc

B = 4
H_DIM = 4096
W_DIM = 1024
OPT_W = 64
WORDS = B * H_DIM * W_DIM
BATCH_WORDS = H_DIM * W_DIM
TILES_PER_BATCH = 8
SLAB = BATCH_WORDS // TILES_PER_BATCH
CHUNK = 65536
N_CHUNK = SLAB // CHUNK
PTS = W_DIM // TILES_PER_BATCH


def _index_body(tlc_ref, tl_ref, h_ref, idx_ref, val_ref):
    col = lax.broadcasted_iota(jnp.int32, (8, 128), 1)
    row = lax.broadcasted_iota(jnp.int32, (8, 128), 0)
    p = row * 128 + col
    gi = p // 32
    gj = p % 32
    gxb = (gi * 4).astype(jnp.float32)
    gyb = (gj * 4).astype(jnp.float32)
    t0 = tl_ref[0]
    t1 = tl_ref[1]
    for b in range(B):
        m00, m01, m02 = h_ref[b, 0, 0], h_ref[b, 0, 1], h_ref[b, 0, 2]
        m10, m11, m12 = h_ref[b, 1, 0], h_ref[b, 1, 1], h_ref[b, 1, 2]
        m20, m21, m22 = h_ref[b, 2, 0], h_ref[b, 2, 1], h_ref[b, 2, 2]
        c00 = m11 * m22 - m12 * m21
        c01 = m12 * m20 - m10 * m22
        c02 = m10 * m21 - m11 * m20
        c10 = m02 * m21 - m01 * m22
        c11 = m00 * m22 - m02 * m20
        c12 = m01 * m20 - m00 * m21
        c20 = m01 * m12 - m02 * m11
        c21 = m02 * m10 - m00 * m12
        c22 = m00 * m11 - m01 * m10
        det = m00 * c00 + m01 * c01 + m02 * c02
        inv = ((c00 / det, c10 / det, c20 / det),
               (c01 / det, c11 / det, c21 / det),
               (c02 / det, c12 / det, c22 / det))
        gx = gxb + tlc_ref[b, 0]
        gy = gyb + tlc_ref[b, 1]
        gyr = gy.astype(jnp.bfloat16).astype(jnp.float32)
        gxr = gx.astype(jnp.bfloat16).astype(jnp.float32)
        ct = []
        for m in range(3):
            i0 = inv[m][0].astype(jnp.bfloat16).astype(jnp.float32)
            i1 = inv[m][1].astype(jnp.bfloat16).astype(jnp.float32)
            i2 = inv[m][2].astype(jnp.bfloat16).astype(jnp.float32)
            ct.append((gyr * i0 + gxr * i1) + i2)
        x = ct[0] / ct[2] + t0
        y = ct[1] / ct[2] + t1
        ihf = jnp.floor(y * 0.125)
        iwf = jnp.floor(x * 0.125)
        flatf = ihf * float(OPT_W) + iwf
        valid = (flatf >= 0.0) & (flatf <= float(H_DIM - 1))
        flati = jnp.where(valid, flatf, 0.0).astype(jnp.int32)
        off = b * BATCH_WORDS + flati * W_DIM + p
        idx_ref[pl.ds(b * 8, 8), :] = off
        val_ref[pl.ds(b * 8, 8), :] = jnp.where(valid, 1.0, 0.0).astype(jnp.float32)


def _compute_offsets(tl_croped, tl, H):
    idx2d, val2d = pl.pallas_call(
        _index_body,
        out_shape=(
            jax.ShapeDtypeStruct((4 * 8, 128), jnp.int32),
            jax.ShapeDtypeStruct((4 * 8, 128), jnp.float32),
        ),
        in_specs=[
            pl.BlockSpec(memory_space=pltpu.SMEM),
            pl.BlockSpec(memory_space=pltpu.SMEM),
            pl.BlockSpec(memory_space=pltpu.SMEM),
        ],
    )(tl_croped, tl, H)
    return idx2d.reshape(B * W_DIM), val2d.reshape(B * W_DIM)


_SC_MESH = plsc.VectorSubcoreMesh(core_axis_name="c", subcore_axis_name="s")


@functools.partial(
    pl.kernel,
    out_type=jax.ShapeDtypeStruct((WORDS,), jnp.float32),
    mesh=_SC_MESH,
    scratch_types=[
        pltpu.VMEM((CHUNK,), jnp.float32),
        pltpu.VMEM((PTS,), jnp.int32),
        pltpu.VMEM((PTS,), jnp.float32),
        pltpu.SemaphoreType.DMA,
        pltpu.SemaphoreType.DMA,
    ],
)
def _scatter_body(idx_hbm, val_hbm, out_hbm, z, idxv, valv, zsem, ssem):
    c = lax.axis_index("c")
    s = lax.axis_index("s")
    b = c * 2 + s // TILES_PER_BATCH
    g = s % TILES_PER_BATCH
    pltpu.sync_copy(idx_hbm.at[pl.ds(b * W_DIM + g * PTS, PTS)], idxv)
    pltpu.sync_copy(val_hbm.at[pl.ds(b * W_DIM + g * PTS, PTS)], valv)

    def zbody(t, carry):
        base = t * 128
        for u in range(8):
            z[pl.ds(base + u * 16, 16)] = jnp.zeros((16,), jnp.float32)
        return carry
    lax.fori_loop(0, CHUNK // 128, zbody, 0)

    slab = b * BATCH_WORDS + g * SLAB
    copies = [
        pltpu.async_copy(z, out_hbm.at[pl.ds(slab + d_ * CHUNK, CHUNK)], zsem)
        for d_ in range(N_CHUNK)
    ]
    for cp in copies:
        cp.wait()
    plsc.subcore_barrier()
    pltpu.async_copy(valv, out_hbm.at[idxv], ssem).wait()


def kernel(tl_croped, tl, H):
    idx1d, val1d = _compute_offsets(tl_croped, tl, H)
    gt = _scatter_body(idx1d, val1d)
    return gt.reshape(B, H_DIM, W_DIM)


# speedup vs baseline: 4.9461x; 1.0171x over previous
import functools

import jax
import jax.numpy as jnp
from jax import lax
from jax.experimental import pallas as pl
from jax.experimental.pallas import tpu as pltpu
from jax.experimental.pallas import tpu_sc as plsc

B = 4
H_DIM = 4096
W_DIM = 1024
OPT_W = 64
WORDS = B * H_DIM * W_DIM
BATCH_WORDS = H_DIM * W_DIM
TILES_PER_BATCH = 8
SLAB = BATCH_WORDS // TILES_PER_BATCH
CHUNK = 65536
N_CHUNK = SLAB // CHUNK
PTS = W_DIM // TILES_PER_BATCH
L = 16


def _rb(v):
    i = lax.bitcast_convert_type(v, jnp.int32)
    bias = jnp.int32(0x7FFF) + ((i >> 16) & 1)
    i2 = lax.bitwise_and(i + bias, jnp.int32(-65536))
    return lax.bitcast_convert_type(i2, jnp.float32)


_SC_MESH = plsc.VectorSubcoreMesh(core_axis_name="c", subcore_axis_name="s")


@functools.partial(
    pl.kernel,
    out_type=jax.ShapeDtypeStruct((WORDS,), jnp.float32),
    mesh=_SC_MESH,
    scratch_types=[
        pltpu.VMEM((16, L), jnp.float32),
        pltpu.VMEM((CHUNK,), jnp.float32),
        pltpu.VMEM((PTS,), jnp.int32),
        pltpu.VMEM((PTS,), jnp.float32),
        pltpu.SemaphoreType.DMA,
        pltpu.SemaphoreType.DMA,
    ],
)
def _gt_body(p_hbm, out_hbm, pv, z, idxv, valv, zsem, ssem):
    c = lax.axis_index("c")
    s = lax.axis_index("s")
    b = c * 2 + s // TILES_PER_BATCH
    g = s % TILES_PER_BATCH
    pltpu.sync_copy(p_hbm.at[b], pv)

    m00, m01, m02 = pv[0, :], pv[1, :], pv[2, :]
    m10, m11, m12 = pv[3, :], pv[4, :], pv[5, :]
    m20, m21, m22 = pv[6, :], pv[7, :], pv[8, :]
    c00 = m11 * m22 - m12 * m21
    c01 = m12 * m20 - m10 * m22
    c02 = m10 * m21 - m11 * m20
    c10 = m02 * m21 - m01 * m22
    c11 = m00 * m22 - m02 * m20
    c12 = m01 * m20 - m00 * m21
    c20 = m01 * m12 - m02 * m11
    c21 = m02 * m10 - m00 * m12
    c22 = m00 * m11 - m01 * m10
    det = m00 * c00 + m01 * c01 + m02 * c02
    i0 = (_rb(c00 / det), _rb(c10 / det), _rb(c20 / det))
    i1 = (_rb(c01 / det), _rb(c11 / det), _rb(c21 / det))
    i2 = (_rb(c02 / det), _rb(c12 / det), _rb(c22 / det))
    i1 = (_rb(c01 / det), _rb(c11 / det), _rb(c21 / det))
    i2 = (_rb(c02 / det), _rb(c12 / det), _rb(c22 / det))
    tc0, tc1 = pv[9, :], pv[10, :]
    t0, t1 = pv[11, :], pv[12, :]
    lane = lax.iota(jnp.int32, L)
    base_w = g * PTS
    for k in range(PTS // L):
        w = base_w + k * L + lane
        gi = lax.shift_right_logical(w, 5)
        gj = lax.bitwise_and(w, 31)
        gx = gi.astype(jnp.float32) * 4.0 + tc0
        gy = gj.astype(jnp.float32) * 4.0 + tc1
        gyr = _rb(gy)
        gxr = _rb(gx)
        ct0 = (gyr * i0[0] + gxr * i0[1]) + i0[2]
        ct1 = (gyr * i1[0] + gxr * i1[1]) + i1[2]
        ct2 = (gyr * i2[0] + gxr * i2[1]) + i2[2]
        x = ct0 / ct2 + t0
        y = ct1 / ct2 + t1
        qy = y * 0.125
        qx = x * 0.125
        ihi = qy.astype(jnp.int32)
        iwi = qx.astype(jnp.int32)
        ihi = ihi - jnp.where(ihi.astype(jnp.float32) > qy, 1, 0)
        iwi = iwi - jnp.where(iwi.astype(jnp.float32) > qx, 1, 0)
        flatf = ihi.astype(jnp.float32) * float(OPT_W) + iwi.astype(jnp.float32)
        valid = (flatf >= 0.0) & (flatf <= float(H_DIM - 1))
        flati = jnp.where(valid, flatf, 0.0).astype(jnp.int32)
        idxv[pl.ds(k * L, L)] = b * BATCH_WORDS + flati * W_DIM + w
        valv[pl.ds(k * L, L)] = jnp.where(valid, 1.0, 0.0).astype(jnp.float32)

    def zbody(t, carry):
        zb = t * 128
        for u in range(8):
            z[pl.ds(zb + u * L, L)] = jnp.zeros((L,), jnp.float32)
        return carry
    lax.fori_loop(0, CHUNK // 128, zbody, 0)

    slab = b * BATCH_WORDS + g * SLAB
    copies = [
        pltpu.async_copy(z, out_hbm.at[pl.ds(slab + d_ * CHUNK, CHUNK)], zsem)
        for d_ in range(N_CHUNK)
    ]
    for cp in copies:
        cp.wait()
    plsc.subcore_barrier()
    pltpu.async_copy(valv, out_hbm.at[idxv], ssem).wait()


def kernel(tl_croped, tl, H):
    params = jnp.concatenate(
        [
            H.reshape(B, 9),
            tl_croped,
            jnp.broadcast_to(tl[None, :], (B, 2)),
            jnp.zeros((B, 3), jnp.float32),
        ],
        axis=1,
    )
    params = jnp.broadcast_to(params[:, :, None], (B, 16, L))
    gt = _gt_body(params)
    return gt.reshape(B, H_DIM, W_DIM)


# speedup vs baseline: 8.3702x; 1.6923x over previous
import functools

import jax
import jax.numpy as jnp
from jax import lax
from jax.experimental import pallas as pl
from jax.experimental.pallas import tpu as pltpu
from jax.experimental.pallas import tpu_sc as plsc

B = 4
H_DIM = 4096
W_DIM = 1024
OPT_W = 64
TILES_PER_BATCH = 8
SLAB_ROWS = H_DIM // TILES_PER_BATCH
ZROWS = 64
N_ZCHUNK = SLAB_ROWS // ZROWS
DROWS = 32
N_DCHUNK = SLAB_ROWS // DROWS
NPTS = W_DIM
L = 16


def _rb(v):
    i = lax.bitcast_convert_type(v, jnp.int32)
    bias = jnp.int32(0x7FFF) + ((i >> 16) & 1)
    i2 = lax.bitwise_and(i + bias, jnp.int32(-65536))
    return lax.bitcast_convert_type(i2, jnp.float32)


_SC_MESH = plsc.VectorSubcoreMesh(core_axis_name="c", subcore_axis_name="s")


@functools.partial(
    pl.kernel,
    out_type=jax.ShapeDtypeStruct((B, H_DIM, W_DIM), jnp.float32),
    mesh=_SC_MESH,
    scratch_types=[
        pltpu.VMEM((16, L), jnp.float32),
        pltpu.VMEM((ZROWS, W_DIM), jnp.float32),
        pltpu.VMEM((DROWS, W_DIM), jnp.float32),
        pltpu.VMEM((NPTS,), jnp.int32),
        pltpu.SemaphoreType.DMA,
    ],
)
def _gt_body(p_hbm, out_hbm, pv, z, zd, flatv, zsem):
    c = lax.axis_index("c")
    s = lax.axis_index("s")
    b = c * 2 + s // TILES_PER_BATCH
    g = s % TILES_PER_BATCH
    slab0 = g * SLAB_ROWS
    pltpu.sync_copy(p_hbm.at[b], pv)

    m00, m01, m02 = pv[0, :], pv[1, :], pv[2, :]
    m10, m11, m12 = pv[3, :], pv[4, :], pv[5, :]
    m20, m21, m22 = pv[6, :], pv[7, :], pv[8, :]
    tc0, tc1 = pv[9, :], pv[10, :]
    t0, t1 = pv[11, :], pv[12, :]
    c00 = m11 * m22 - m12 * m21
    c01 = m12 * m20 - m10 * m22
    c02 = m10 * m21 - m11 * m20
    c10 = m02 * m21 - m01 * m22
    c11 = m00 * m22 - m02 * m20
    c12 = m01 * m20 - m00 * m21
    c20 = m01 * m12 - m02 * m11
    c21 = m02 * m10 - m00 * m12
    c22 = m00 * m11 - m01 * m10
    det = m00 * c00 + m01 * c01 + m02 * c02
    i0 = (_rb(c00 / det), _rb(c10 / det), _rb(c20 / det))
    i1 = (_rb(c01 / det), _rb(c11 / det), _rb(c21 / det))
    i2 = (_rb(c02 / det), _rb(c12 / det), _rb(c22 / det))

    lane = lax.iota(jnp.int32, L)

    def ptbody(kk, acc):
        w = kk * L + lane
        gi = lax.shift_right_logical(w, 5)
        gj = lax.bitwise_and(w, 31)
        gx = gi.astype(jnp.float32) * 4.0 + tc0
        gy = gj.astype(jnp.float32) * 4.0 + tc1
        gyr = _rb(gy)
        gxr = _rb(gx)
        ct0 = (gyr * i0[0] + gxr * i0[1]) + i0[2]
        ct1 = (gyr * i1[0] + gxr * i1[1]) + i1[2]
        ct2 = (gyr * i2[0] + gxr * i2[1]) + i2[2]
        x = ct0 / ct2 + t0
        y = ct1 / ct2 + t1
        qy = y * 0.125
        qx = x * 0.125
        ihi = qy.astype(jnp.int32)
        iwi = qx.astype(jnp.int32)
        ihi = ihi - jnp.where(ihi.astype(jnp.float32) > qy, 1, 0)
        iwi = iwi - jnp.where(iwi.astype(jnp.float32) > qx, 1, 0)
        flatf = ihi.astype(jnp.float32) * float(OPT_W) + iwi.astype(jnp.float32)
        valid = (flatf >= 0.0) & (flatf <= float(H_DIM - 1))
        flati = jnp.where(valid, flatf, -1.0).astype(jnp.int32)
        local = flati - slab0
        flatv[pl.ds(kk * L, L)] = local
        in_slab = (local >= 0) & (local < SLAB_ROWS)
        cid = lax.bitwise_and(lax.shift_right_logical(local, 5), 15)
        bits = jnp.where(in_slab, jnp.left_shift(1, cid), 0)
        return acc | bits

    acc = lax.fori_loop(0, NPTS // L, ptbody, jnp.zeros((L,), jnp.int32))

    def zbody(t, carry):
        r = lax.shift_right_logical(t, 3)
        cb = lax.bitwise_and(t, 7) * 128
        for u in range(8):
            z[r, pl.ds(cb + u * L, L)] = jnp.zeros((L,), jnp.float32)
        return carry
    lax.fori_loop(0, 512, zbody, 0)

    copies = [
        pltpu.async_copy(
            z, out_hbm.at[b, pl.ds(slab0 + d_ * ZROWS, ZROWS), :], zsem)
        for d_ in range(N_ZCHUNK)
    ]
    for cp in copies:
        cp.wait()

    bitmap = acc[0]
    for ll in range(1, L):
        bitmap = bitmap | acc[ll]
    for cc in range(N_DCHUNK):
        dirty = lax.bitwise_and(bitmap, jnp.int32(1 << cc)) > 0

        @pl.when(dirty)
        def _(cc=cc):
            def sbody(kk, carry):
                rel = flatv[pl.ds(kk * L, L)] - cc * DROWS
                for r in range(DROWS):
                    zd[r, pl.ds(kk * L, L)] = jnp.where(
                        rel == r, 1.0, 0.0).astype(jnp.float32)
                return carry
            lax.fori_loop(0, NPTS // L, sbody, 0)
            pltpu.sync_copy(
                zd, out_hbm.at[b, pl.ds(slab0 + cc * DROWS, DROWS), :])


def kernel(tl_croped, tl, H):
    params = jnp.concatenate(
        [
            H.reshape(B, 9),
            tl_croped,
            jnp.broadcast_to(tl[None, :], (B, 2)),
            jnp.zeros((B, 3), jnp.float32),
        ],
        axis=1,
    )
    params = jnp.broadcast_to(params[:, :, None], (B, 16, L))
    return _gt_body(params)


# speedup vs baseline: 8.5002x; 1.0155x over previous
import functools

import jax
import jax.numpy as jnp
from jax import lax
from jax.experimental import pallas as pl
from jax.experimental.pallas import tpu as pltpu
from jax.experimental.pallas import tpu_sc as plsc

B = 4
H_DIM = 4096
W_DIM = 1024
OPT_W = 64
TILES_PER_BATCH = 8
SLAB_ROWS = H_DIM // TILES_PER_BATCH
ZROWS = 64
N_ZCHUNK = SLAB_ROWS // ZROWS
DROWS = 32
N_DCHUNK = SLAB_ROWS // DROWS
NPTS = W_DIM
L = 16


def _rb(v):
    i = lax.bitcast_convert_type(v, jnp.int32)
    bias = jnp.int32(0x7FFF) + ((i >> 16) & 1)
    i2 = lax.bitwise_and(i + bias, jnp.int32(-65536))
    return lax.bitcast_convert_type(i2, jnp.float32)


_SC_MESH = plsc.VectorSubcoreMesh(core_axis_name="c", subcore_axis_name="s")


@functools.partial(
    pl.kernel,
    out_type=jax.ShapeDtypeStruct((B, H_DIM, W_DIM), jnp.float32),
    mesh=_SC_MESH,
    scratch_types=[
        pltpu.VMEM((16, L), jnp.float32),
        pltpu.VMEM((ZROWS, W_DIM), jnp.float32),
        pltpu.VMEM((DROWS, W_DIM), jnp.float32),
        pltpu.VMEM((NPTS,), jnp.int32),
        pltpu.SemaphoreType.DMA,
    ],
)
def _gt_body(p_hbm, out_hbm, pv, z, zd, flatv, zsem):
    c = lax.axis_index("c")
    s = lax.axis_index("s")
    b = c * 2 + s // TILES_PER_BATCH
    g = s % TILES_PER_BATCH
    slab0 = g * SLAB_ROWS
    pltpu.sync_copy(p_hbm.at[b], pv)

    m00, m01, m02 = pv[0, :], pv[1, :], pv[2, :]
    m10, m11, m12 = pv[3, :], pv[4, :], pv[5, :]
    m20, m21, m22 = pv[6, :], pv[7, :], pv[8, :]
    tc0, tc1 = pv[9, :], pv[10, :]
    t0, t1 = pv[11, :], pv[12, :]
    c00 = m11 * m22 - m12 * m21
    c01 = m12 * m20 - m10 * m22
    c02 = m10 * m21 - m11 * m20
    c10 = m02 * m21 - m01 * m22
    c11 = m00 * m22 - m02 * m20
    c12 = m01 * m20 - m00 * m21
    c20 = m01 * m12 - m02 * m11
    c21 = m02 * m10 - m00 * m12
    c22 = m00 * m11 - m01 * m10
    det = m00 * c00 + m01 * c01 + m02 * c02
    i0 = (_rb(c00 / det), _rb(c10 / det), _rb(c20 / det))
    i1 = (_rb(c01 / det), _rb(c11 / det), _rb(c21 / det))
    i2 = (_rb(c02 / det), _rb(c12 / det), _rb(c22 / det))

    def zbody(t, carry):
        r = lax.shift_right_logical(t, 3)
        cb = lax.bitwise_and(t, 7) * 128
        for u in range(8):
            z[r, pl.ds(cb + u * L, L)] = jnp.zeros((L,), jnp.float32)
        return carry
    lax.fori_loop(0, 512, zbody, 0)
    copies = [
        pltpu.async_copy(
            z, out_hbm.at[b, pl.ds(slab0 + d_ * ZROWS, ZROWS), :], zsem)
        for d_ in range(N_ZCHUNK)
    ]

    lane = lax.iota(jnp.int32, L)

    def ptbody(kk, acc):
        w = kk * L + lane
        gi = lax.shift_right_logical(w, 5)
        gj = lax.bitwise_and(w, 31)
        gx = gi.astype(jnp.float32) * 4.0 + tc0
        gy = gj.astype(jnp.float32) * 4.0 + tc1
        gyr = _rb(gy)
        gxr = _rb(gx)
        ct0 = (gyr * i0[0] + gxr * i0[1]) + i0[2]
        ct1 = (gyr * i1[0] + gxr * i1[1]) + i1[2]
        ct2 = (gyr * i2[0] + gxr * i2[1]) + i2[2]
        x = ct0 / ct2 + t0
        y = ct1 / ct2 + t1
        qy = y * 0.125
        qx = x * 0.125
        ihi = qy.astype(jnp.int32)
        iwi = qx.astype(jnp.int32)
        ihi = ihi - jnp.where(ihi.astype(jnp.float32) > qy, 1, 0)
        iwi = iwi - jnp.where(iwi.astype(jnp.float32) > qx, 1, 0)
        flatf = ihi.astype(jnp.float32) * float(OPT_W) + iwi.astype(jnp.float32)
        valid = (flatf >= 0.0) & (flatf <= float(H_DIM - 1))
        flati = jnp.where(valid, flatf, -1.0).astype(jnp.int32)
        local = flati - slab0
        flatv[pl.ds(kk * L, L)] = local
        in_slab = (local >= 0) & (local < SLAB_ROWS)
        cid = lax.bitwise_and(lax.shift_right_logical(local, 5), 15)
        bits = jnp.where(in_slab, jnp.left_shift(1, cid), 0)
        return acc | bits

    acc = lax.fori_loop(0, NPTS // L, ptbody, jnp.zeros((L,), jnp.int32))

    for cp in copies:
        cp.wait()

    bitmap = acc[0]
    for ll in range(1, L):
        bitmap = bitmap | acc[ll]
    for cc in range(N_DCHUNK):
        dirty = lax.bitwise_and(bitmap, jnp.int32(1 << cc)) > 0

        @pl.when(dirty)
        def _(cc=cc):
            def sbody(kk, carry):
                rel = flatv[pl.ds(kk * L, L)] - cc * DROWS
                for r in range(DROWS):
                    zd[r, pl.ds(kk * L, L)] = jnp.where(
                        rel == r, 1.0, 0.0).astype(jnp.float32)
                return carry
            lax.fori_loop(0, NPTS // L, sbody, 0)
            pltpu.sync_copy(
                zd, out_hbm.at[b, pl.ds(slab0 + cc * DROWS, DROWS), :])


def kernel(tl_croped, tl, H):
    params = jnp.concatenate(
        [
            H.reshape(B, 9),
            tl_croped,
            jnp.broadcast_to(tl[None, :], (B, 2)),
            jnp.zeros((B, 3), jnp.float32),
        ],
        axis=1,
    )
    params = jnp.broadcast_to(params[:, :, None], (B, 16, L))
    return _gt_body(params)
